# baseline (speedup 1.0000x reference)
import functools

import jax
import jax.numpy as jnp
from jax import lax
from jax.experimental import pallas as pl
from jax.experimental.pallas import tpu as pltpu
from jax.experimental.pallas import tpu_sc as plsc

_NUM_TABLES = 1000
_NUM_LAYER = 12
_BATCH = 4096
_D = _NUM_LAYER * 32 * 32
_NC, _NS = 2, 16
_NW = _NC * _NS
_K = 4
_SC_ROWS = 2560
_TC_ROWS = _BATCH - _SC_ROWS
_NSEM = 8
_WIN = 32


def _make_sc_gather(batch):
    b_per_w = batch // _NW
    n_chunk = b_per_w // _K
    mesh = plsc.VectorSubcoreMesh(core_axis_name="c", subcore_axis_name="s")

    @functools.partial(
        pl.kernel,
        mesh=mesh,
        out_type=jax.ShapeDtypeStruct((batch, _D), jnp.float32),
        scratch_types=[
            pltpu.VMEM((n_chunk, _K), jnp.int32),
            pltpu.VMEM((_K, _D), jnp.float32),
            pltpu.VMEM((_K, _D), jnp.float32),
            pltpu.SemaphoreType.DMA,
            pltpu.SemaphoreType.DMA,
            pltpu.SemaphoreType.DMA,
            pltpu.SemaphoreType.DMA,
        ],
    )
    def sc_gather(idx_hbm, table_hbm, out_hbm, idx_v,
                  buf0, buf1, gsem0, gsem1, osem0, osem1):
        wid = lax.axis_index("s") * _NC + lax.axis_index("c")
        pltpu.sync_copy(idx_hbm.at[wid], idx_v)
        base = wid * b_per_w
        bufs = (buf0, buf1)
        gsems = (gsem0, gsem1)
        osems = (osem0, osem1)

        def wait_gather(p):
            pltpu.make_async_copy(
                table_hbm.at[idx_v.at[0]], bufs[p], gsems[p]).wait()

        def wait_out(p):
            pltpu.make_async_copy(
                bufs[p], out_hbm.at[pl.ds(0, _K)], osems[p]).wait()

        def start_gather(j, p):
            pltpu.async_copy(table_hbm.at[idx_v.at[j]], bufs[p], gsems[p])

        def start_out(j, p):
            pltpu.async_copy(bufs[p], out_hbm.at[pl.ds(base + j * _K, _K)],
                             osems[p])

        start_gather(0, 0)
        wait_gather(0)
        start_out(0, 0)
        start_gather(1, 1)

        def body(i, carry):
            j0 = 2 * i + 1
            wait_gather(1)
            start_out(j0, 1)
            wait_out(0)
            start_gather(j0 + 1, 0)
            wait_gather(0)
            start_out(j0 + 1, 0)
            wait_out(1)
            start_gather(j0 + 2, 1)
            return carry

        lax.fori_loop(0, n_chunk // 2 - 1, body, 0)

        j_last = n_chunk - 1
        wait_gather(1)
        start_out(j_last, 1)
        wait_out(0)
        wait_out(1)

    return sc_gather


_sc_gather = _make_sc_gather(_SC_ROWS)


def _tc_body(idx_ref, table_ref, out_ref, *sems):
    def start(j, p):
        pltpu.make_async_copy(
            table_ref.at[pl.ds(idx_ref[j], 1)],
            out_ref.at[pl.ds(j, 1)],
            sems[p],
        ).start()

    def wait(j, p):
        pltpu.make_async_copy(
            table_ref.at[pl.ds(0, 1)],
            out_ref.at[pl.ds(j, 1)],
            sems[p],
        ).wait()

    def body(i, carry):
        for p in range(_NSEM):
            j = i * _NSEM + p
            start(j, p)

            @pl.when(j >= _WIN)
            def _():
                wait(j - _WIN, p)

        return carry

    lax.fori_loop(0, _TC_ROWS // _NSEM, body, 0)

    for j in range(_TC_ROWS - _WIN, _TC_ROWS):
        wait(j, j % _NSEM)


def _tc_gather(idx_tc, table2):
    grid_spec = pltpu.PrefetchScalarGridSpec(
        num_scalar_prefetch=1,
        grid=(1,),
        in_specs=[pl.BlockSpec(memory_space=pl.ANY)],
        out_specs=pl.BlockSpec(memory_space=pl.ANY),
        scratch_shapes=[pltpu.SemaphoreType.DMA] * _NSEM,
    )
    return pl.pallas_call(
        _tc_body,
        grid_spec=grid_spec,
        out_shape=jax.ShapeDtypeStruct((_TC_ROWS, _D), jnp.float32),
    )(idx_tc, table2)


def kernel(img_index, table):
    table2 = table.reshape(_NUM_TABLES, _D)
    idx_sc = lax.slice(img_index, (0,), (_SC_ROWS,)).reshape(
        _NW, _SC_ROWS // _NW // _K, _K)
    idx_tc = lax.slice(img_index, (_SC_ROWS,), (_BATCH,))
    out_sc = _sc_gather(idx_sc, table2)
    out_tc = _tc_gather(idx_tc, table2)
    out2 = jnp.concatenate([out_sc, out_tc], axis=0)
    return out2.reshape(_BATCH, _NUM_LAYER, 32, 32)
